# baseline (speedup 1.0000x reference)
import functools

import jax
import jax.numpy as jnp
from jax import lax
from jax.experimental import pallas as pl
from jax.experimental.pallas import tpu as pltpu
from jax.experimental.pallas import tpu_sc as plsc


def kernel(user, item, user_table, item_table):
    B = user.shape[0]
    N, D = user_table.shape
    Np = N + (-N % 8)

    info = plsc.get_sparse_core_info()
    NC, NS, L = info.num_cores, info.num_subcores, info.num_lanes
    NW = NC * NS
    bpw = B // NW

    @functools.partial(
        pl.kernel,
        out_type=jax.ShapeDtypeStruct((B,), jnp.float32),
        mesh=plsc.VectorSubcoreMesh(core_axis_name="c", subcore_axis_name="s"),
        compiler_params=pltpu.CompilerParams(use_tc_tiling_on_sc=False),
        scratch_types=[
            pltpu.VMEM((bpw,), jnp.int32),
            pltpu.VMEM((bpw,), jnp.int32),
            pltpu.VMEM((D, bpw), jnp.float32),
            pltpu.VMEM((D, bpw), jnp.float32),
            pltpu.VMEM((bpw,), jnp.float32),
            pltpu.SemaphoreType.DMA,
        ],
    )
    def mf_kernel(user_hbm, item_hbm, ut_hbm, it_hbm, out_hbm,
                  uidx_v, iidx_v, ubuf_v, ibuf_v, out_v, sem):
        wid = lax.axis_index("s") * NC + lax.axis_index("c")
        base = wid * bpw

        pltpu.sync_copy(user_hbm.at[pl.ds(base, bpw)], uidx_v)
        pltpu.sync_copy(item_hbm.at[pl.ds(base, bpw)], iidx_v)

        for d in range(D):
            pltpu.async_copy(
                ut_hbm.at[pl.ds(d * Np, Np)].at[uidx_v], ubuf_v.at[d], sem
            )
            pltpu.async_copy(
                it_hbm.at[pl.ds(d * Np, Np)].at[iidx_v], ibuf_v.at[d], sem
            )
        for d in range(D):
            pltpu.make_async_copy(
                ut_hbm.at[pl.ds(0, bpw)], ubuf_v.at[d], sem
            ).wait()
            pltpu.make_async_copy(
                it_hbm.at[pl.ds(0, bpw)], ibuf_v.at[d], sem
            ).wait()

        def body(j, carry):
            sl = pl.ds(j * L, L)
            acc = jnp.zeros((L,), jnp.float32)
            for d in range(D):
                acc = acc + ubuf_v[d, sl] * ibuf_v[d, sl]
            out_v[sl] = acc
            return carry

        lax.fori_loop(0, bpw // L, body, 0)

        pltpu.sync_copy(out_v, out_hbm.at[pl.ds(base, bpw)])

    ut_flat = jnp.pad(user_table.T, ((0, 0), (0, Np - N))).reshape(-1)
    it_flat = jnp.pad(item_table.T, ((0, 0), (0, Np - N))).reshape(-1)
    return mf_kernel(user, item, ut_flat, it_flat)


# speedup vs baseline: 1.0291x; 1.0291x over previous
import functools

import jax
import jax.numpy as jnp
from jax import lax
from jax.experimental import pallas as pl
from jax.experimental.pallas import tpu as pltpu
from jax.experimental.pallas import tpu_sc as plsc


def kernel(user, item, user_table, item_table):
    B = user.shape[0]
    N, D = user_table.shape
    Np = N + (-N % 8)

    info = plsc.get_sparse_core_info()
    NC, NS, L = info.num_cores, info.num_subcores, info.num_lanes
    NW = NC * NS
    bpw = B // NW

    @functools.partial(
        pl.kernel,
        out_type=jax.ShapeDtypeStruct((B,), jnp.float32),
        mesh=plsc.VectorSubcoreMesh(core_axis_name="c", subcore_axis_name="s"),
        compiler_params=pltpu.CompilerParams(use_tc_tiling_on_sc=False),
        scratch_types=[
            pltpu.VMEM((bpw,), jnp.int32),
            pltpu.VMEM((bpw,), jnp.int32),
            pltpu.VMEM((D, bpw), jnp.float32),
            pltpu.VMEM((D, bpw), jnp.float32),
            pltpu.VMEM((bpw,), jnp.float32),
            pltpu.SemaphoreType.DMA,
        ],
    )
    def mf_kernel(user_hbm, item_hbm, ut_hbm, it_hbm, out_hbm,
                  uidx_v, iidx_v, ubuf_v, ibuf_v, out_v, sem):
        wid = lax.axis_index("s") * NC + lax.axis_index("c")
        base = wid * bpw

        pltpu.sync_copy(user_hbm.at[pl.ds(base, bpw)], uidx_v)
        pltpu.sync_copy(item_hbm.at[pl.ds(base, bpw)], iidx_v)

        for d in range(D):
            pltpu.async_copy(ut_hbm.at[d].at[uidx_v], ubuf_v.at[d], sem)
            pltpu.async_copy(it_hbm.at[d].at[iidx_v], ibuf_v.at[d], sem)
        for d in range(D):
            pltpu.make_async_copy(
                ut_hbm.at[0].at[pl.ds(0, bpw)], ubuf_v.at[d], sem
            ).wait()
            pltpu.make_async_copy(
                it_hbm.at[0].at[pl.ds(0, bpw)], ibuf_v.at[d], sem
            ).wait()

        def body(j, carry):
            sl = pl.ds(j * L, L)
            acc = jnp.zeros((L,), jnp.float32)
            for d in range(D):
                acc = acc + ubuf_v[d, sl] * ibuf_v[d, sl]
            out_v[sl] = acc
            return carry

        lax.fori_loop(0, bpw // L, body, 0)

        pltpu.sync_copy(out_v, out_hbm.at[pl.ds(base, bpw)])

    N0 = N - (N % 8)
    ut_t, it_t = lax.optimization_barrier(
        (user_table.T[:, :N0], item_table.T[:, :N0])
    )
    return mf_kernel(user, item, ut_t, it_t)


# speedup vs baseline: 5.8176x; 5.6532x over previous
import functools

import jax
import jax.numpy as jnp
from jax import lax
from jax.experimental import pallas as pl
from jax.experimental.pallas import tpu as pltpu
from jax.experimental.pallas import tpu_sc as plsc


def kernel(user, item, user_table, item_table):
    B = user.shape[0]
    D = user_table.shape[1]

    info = plsc.get_sparse_core_info()
    NC, NS, L = info.num_cores, info.num_subcores, info.num_lanes
    NW = NC * NS
    bpw = B // NW

    @functools.partial(
        pl.kernel,
        out_type=jax.ShapeDtypeStruct((B,), jnp.float32),
        mesh=plsc.VectorSubcoreMesh(core_axis_name="c", subcore_axis_name="s"),
        compiler_params=pltpu.CompilerParams(
            use_tc_tiling_on_sc=False, needs_layout_passes=False
        ),
        scratch_types=[
            pltpu.VMEM((bpw,), jnp.int32),
            pltpu.VMEM((bpw,), jnp.int32),
            pltpu.VMEM((bpw, D), jnp.float32),
            pltpu.VMEM((bpw, D), jnp.float32),
            pltpu.VMEM((bpw,), jnp.float32),
            pltpu.SemaphoreType.DMA,
        ],
    )
    def mf_kernel(user_hbm, item_hbm, ut_hbm, it_hbm, out_hbm,
                  uidx_v, iidx_v, urows_v, irows_v, out_v, sem):
        wid = lax.axis_index("s") * NC + lax.axis_index("c")
        base = wid * bpw

        pltpu.sync_copy(user_hbm.at[pl.ds(base, bpw)], uidx_v)
        pltpu.sync_copy(item_hbm.at[pl.ds(base, bpw)], iidx_v)

        cu = pltpu.async_copy(ut_hbm.at[uidx_v], urows_v, sem)
        ci = pltpu.async_copy(it_hbm.at[iidx_v], irows_v, sem)
        cu.wait()
        ci.wait()

        lanes = lax.iota(jnp.int32, L)

        def body(g, carry):
            row = g * L + lanes
            acc = jnp.zeros((L,), jnp.float32)
            for d in range(D):
                col = jnp.full((L,), d, jnp.int32)
                uu = plsc.load_gather(urows_v, [row, col])
                ii = plsc.load_gather(irows_v, [row, col])
                acc = acc + uu * ii
            out_v[pl.ds(g * L, L)] = acc
            return carry

        lax.fori_loop(0, bpw // L, body, 0)

        pltpu.sync_copy(out_v, out_hbm.at[pl.ds(base, bpw)])

    return mf_kernel(user, item, user_table, item_table)


# speedup vs baseline: 10.3643x; 1.7815x over previous
import functools

import jax
import jax.numpy as jnp
from jax import lax
from jax.experimental import pallas as pl
from jax.experimental.pallas import tpu as pltpu
from jax.experimental.pallas import tpu_sc as plsc


def kernel(user, item, user_table, item_table):
    B = user.shape[0]
    N, D = user_table.shape

    info = plsc.get_sparse_core_info()
    NC, NS, L = info.num_cores, info.num_subcores, info.num_lanes
    NW = NC * NS
    bpw = B // NW

    TILE = 128
    SUB = 8
    full_tiles = (N - 1) // TILE
    tail = (N - 1) - full_tiles * TILE
    t_stride = full_tiles + 1
    n_dd = D // SUB
    T3 = n_dd * t_stride
    GRP = 32
    n_grp = -(-full_tiles // (NW * GRP))

    @functools.partial(
        pl.kernel,
        out_type=(
            jax.ShapeDtypeStruct((T3, SUB, TILE), jnp.float32),
            jax.ShapeDtypeStruct((T3, SUB, TILE), jnp.float32),
        ),
        mesh=plsc.VectorSubcoreMesh(core_axis_name="c", subcore_axis_name="s"),
        scratch_types=[
            pltpu.VMEM((GRP, SUB, TILE), jnp.float32),
            pltpu.SemaphoreType.DMA,
            pltpu.SemaphoreType.DMA,
        ],
    )
    def rebase_kernel(ut_hbm, it_hbm, tu_hbm, ti_hbm, uo_hbm, io_hbm,
                      buf_v, semr, semw):
        wid = lax.axis_index("s") * NC + lax.axis_index("c")

        for src, dst in ((ut_hbm, uo_hbm), (it_hbm, io_hbm)):
            for dd in range(n_dd):
                def group(g, carry):
                    for j in range(GRP):
                        jt = wid + (g * GRP + j) * NW
                        jc = jnp.where(jt < full_tiles, jt, 0)
                        col = pl.multiple_of(jc * TILE, TILE)
                        pltpu.async_copy(
                            src.at[pl.ds(dd * SUB, SUB), pl.ds(col, TILE)],
                            buf_v.at[j], semr,
                        )
                    pltpu.make_async_copy(
                        dst.at[pl.ds(0, GRP)], buf_v, semr
                    ).wait()
                    for j in range(GRP):
                        jt = wid + (g * GRP + j) * NW
                        jc = jnp.where(jt < full_tiles, jt, 0)
                        pltpu.async_copy(
                            buf_v.at[j], dst.at[dd * t_stride + jc], semw
                        )
                    pltpu.make_async_copy(
                        dst.at[pl.ds(0, GRP)], buf_v, semw
                    ).wait()
                    return carry

                lax.fori_loop(0, n_grp, group, 0)

        @pl.when(wid == 0)
        def _():
            for tsrc, dst in ((tu_hbm, uo_hbm), (ti_hbm, io_hbm)):
                pltpu.async_copy(tsrc, buf_v.at[pl.ds(0, n_dd)], semr)
                pltpu.make_async_copy(
                    tsrc, buf_v.at[pl.ds(0, n_dd)], semr
                ).wait()
                for dd in range(n_dd):
                    pltpu.async_copy(
                        buf_v.at[dd], dst.at[dd * t_stride + full_tiles],
                        semw,
                    )
                pltpu.make_async_copy(
                    tsrc, buf_v.at[pl.ds(0, n_dd)], semw
                ).wait()

    flat_len = T3 * SUB * TILE
    col_len = full_tiles * SUB * TILE + TILE

    @functools.partial(
        pl.kernel,
        out_type=jax.ShapeDtypeStruct((B,), jnp.float32),
        mesh=plsc.VectorSubcoreMesh(core_axis_name="c", subcore_axis_name="s"),
        compiler_params=pltpu.CompilerParams(use_tc_tiling_on_sc=False),
        scratch_types=[
            pltpu.VMEM((bpw,), jnp.int32),
            pltpu.VMEM((bpw,), jnp.int32),
            pltpu.VMEM((bpw,), jnp.int32),
            pltpu.VMEM((bpw,), jnp.int32),
            pltpu.VMEM((D, bpw), jnp.float32),
            pltpu.VMEM((D, bpw), jnp.float32),
            pltpu.VMEM((bpw,), jnp.float32),
            pltpu.SemaphoreType.DMA,
        ],
    )
    def mf_kernel(user_hbm, item_hbm, ut_hbm, it_hbm, out_hbm,
                  uidx_v, iidx_v, uq_v, iq_v, ubuf_v, ibuf_v, out_v, sem):
        wid = lax.axis_index("s") * NC + lax.axis_index("c")
        base = wid * bpw

        pltpu.sync_copy(user_hbm.at[pl.ds(base, bpw)], uidx_v)
        pltpu.sync_copy(item_hbm.at[pl.ds(base, bpw)], iidx_v)

        def qify(idx_ref, q_ref, g, carry):
            sl = pl.ds(g * L, L)
            r = idx_ref[sl]
            q_ref[sl] = jnp.left_shift(
                jnp.right_shift(r, 7), 10
            ) + jnp.bitwise_and(r, 127)
            return carry

        lax.fori_loop(0, bpw // L, functools.partial(qify, uidx_v, uq_v), 0)
        lax.fori_loop(0, bpw // L, functools.partial(qify, iidx_v, iq_v), 0)

        for d in range(D):
            cbase = (d // SUB) * (t_stride * SUB * TILE) + (d % SUB) * TILE
            pltpu.async_copy(
                ut_hbm.at[pl.ds(cbase, col_len)].at[uq_v], ubuf_v.at[d], sem
            )
            pltpu.async_copy(
                it_hbm.at[pl.ds(cbase, col_len)].at[iq_v], ibuf_v.at[d], sem
            )
        for d in range(D):
            pltpu.make_async_copy(
                ut_hbm.at[pl.ds(0, bpw)], ubuf_v.at[d], sem
            ).wait()
            pltpu.make_async_copy(
                it_hbm.at[pl.ds(0, bpw)], ibuf_v.at[d], sem
            ).wait()

        def body(j, carry):
            sl = pl.ds(j * L, L)
            acc = jnp.zeros((L,), jnp.float32)
            for d in range(D):
                acc = acc + ubuf_v[d, sl] * ibuf_v[d, sl]
            out_v[sl] = acc
            return carry

        lax.fori_loop(0, bpw // L, body, 0)

        pltpu.sync_copy(out_v, out_hbm.at[pl.ds(base, bpw)])

    def tail_tiles(t):
        tl = t.T[:, full_tiles * TILE:N - 1]
        return jnp.pad(tl, ((0, 0), (0, TILE - tail))).reshape(
            n_dd, SUB, TILE)

    u3, i3 = rebase_kernel(
        user_table.T, item_table.T,
        tail_tiles(user_table), tail_tiles(item_table),
    )
    return mf_kernel(user, item, u3.reshape(flat_len), i3.reshape(flat_len))


# speedup vs baseline: 10.5294x; 1.0159x over previous
import functools

import jax
import jax.numpy as jnp
from jax import lax
from jax.experimental import pallas as pl
from jax.experimental.pallas import tpu as pltpu
from jax.experimental.pallas import tpu_sc as plsc


def kernel(user, item, user_table, item_table):
    B = user.shape[0]
    N, D = user_table.shape

    info = plsc.get_sparse_core_info()
    NC, NS, L = info.num_cores, info.num_subcores, info.num_lanes
    NW = NC * NS
    bpw = B // NW

    TILE = 128
    SUB = 8
    full_tiles = (N - 1) // TILE
    tail = (N - 1) - full_tiles * TILE
    t_stride = full_tiles + 1
    n_dd = D // SUB
    T3 = n_dd * t_stride
    GRP = 32
    n_grp = -(-full_tiles // (NW * GRP))

    @functools.partial(
        pl.kernel,
        out_type=(
            jax.ShapeDtypeStruct((T3, SUB, TILE), jnp.float32),
            jax.ShapeDtypeStruct((T3, SUB, TILE), jnp.float32),
        ),
        mesh=plsc.VectorSubcoreMesh(core_axis_name="c", subcore_axis_name="s"),
        scratch_types=[
            pltpu.VMEM((2, GRP, SUB, TILE), jnp.float32),
            pltpu.SemaphoreType.DMA,
            pltpu.SemaphoreType.DMA,
        ],
    )
    def rebase_kernel(ut_hbm, it_hbm, tu_hbm, ti_hbm, uo_hbm, io_hbm,
                      buf_v, semr, semw):
        wid = lax.axis_index("s") * NC + lax.axis_index("c")

        for src, dst in ((ut_hbm, uo_hbm), (it_hbm, io_hbm)):
            for dd in range(n_dd):
                def group(k, carry):
                    @pl.when(k > 0)
                    def _():
                        pltpu.make_async_copy(
                            dst.at[pl.ds(0, GRP)], buf_v.at[0], semw
                        ).wait()
                        pltpu.make_async_copy(
                            dst.at[pl.ds(0, GRP)], buf_v.at[1], semw
                        ).wait()
                    for s in range(2):
                        g = 2 * k + s
                        for j in range(GRP):
                            jt = wid + (g * GRP + j) * NW
                            jc = jnp.where(jt < full_tiles, jt, 0)
                            col = pl.multiple_of(jc * TILE, TILE)
                            pltpu.async_copy(
                                src.at[pl.ds(dd * SUB, SUB),
                                       pl.ds(col, TILE)],
                                buf_v.at[s].at[j], semr,
                            )
                        pltpu.make_async_copy(
                            dst.at[pl.ds(0, GRP)], buf_v.at[s], semr
                        ).wait()
                        for j in range(GRP):
                            jt = wid + (g * GRP + j) * NW
                            jc = jnp.where(jt < full_tiles, jt, 0)
                            pltpu.async_copy(
                                buf_v.at[s].at[j],
                                dst.at[dd * t_stride + jc], semw,
                            )
                    return carry

                lax.fori_loop(0, n_grp // 2, group, 0)
                pltpu.make_async_copy(
                    dst.at[pl.ds(0, GRP)], buf_v.at[0], semw
                ).wait()
                pltpu.make_async_copy(
                    dst.at[pl.ds(0, GRP)], buf_v.at[1], semw
                ).wait()

        @pl.when(wid == 0)
        def _():
            for tsrc, dst in ((tu_hbm, uo_hbm), (ti_hbm, io_hbm)):
                pltpu.async_copy(tsrc, buf_v.at[0].at[pl.ds(0, n_dd)], semr)
                pltpu.make_async_copy(
                    tsrc, buf_v.at[0].at[pl.ds(0, n_dd)], semr
                ).wait()
                for dd in range(n_dd):
                    pltpu.async_copy(
                        buf_v.at[0].at[dd],
                        dst.at[dd * t_stride + full_tiles], semw,
                    )
                pltpu.make_async_copy(
                    tsrc, buf_v.at[0].at[pl.ds(0, n_dd)], semw
                ).wait()

    flat_len = T3 * SUB * TILE
    col_len = full_tiles * SUB * TILE + TILE

    @functools.partial(
        pl.kernel,
        out_type=jax.ShapeDtypeStruct((B,), jnp.float32),
        mesh=plsc.VectorSubcoreMesh(core_axis_name="c", subcore_axis_name="s"),
        compiler_params=pltpu.CompilerParams(use_tc_tiling_on_sc=False),
        scratch_types=[
            pltpu.VMEM((bpw,), jnp.int32),
            pltpu.VMEM((bpw,), jnp.int32),
            pltpu.VMEM((bpw,), jnp.int32),
            pltpu.VMEM((bpw,), jnp.int32),
            pltpu.VMEM((D, bpw), jnp.float32),
            pltpu.VMEM((D, bpw), jnp.float32),
            pltpu.VMEM((bpw,), jnp.float32),
            pltpu.SemaphoreType.DMA,
        ],
    )
    def mf_kernel(user_hbm, item_hbm, ut_hbm, it_hbm, out_hbm,
                  uidx_v, iidx_v, uq_v, iq_v, ubuf_v, ibuf_v, out_v, sem):
        wid = lax.axis_index("s") * NC + lax.axis_index("c")
        base = wid * bpw

        pltpu.sync_copy(user_hbm.at[pl.ds(base, bpw)], uidx_v)
        pltpu.sync_copy(item_hbm.at[pl.ds(base, bpw)], iidx_v)

        def qify(idx_ref, q_ref, g, carry):
            sl = pl.ds(g * L, L)
            r = idx_ref[sl]
            q_ref[sl] = jnp.left_shift(
                jnp.right_shift(r, 7), 10
            ) + jnp.bitwise_and(r, 127)
            return carry

        lax.fori_loop(0, bpw // L, functools.partial(qify, uidx_v, uq_v), 0)
        lax.fori_loop(0, bpw // L, functools.partial(qify, iidx_v, iq_v), 0)

        for d in range(D):
            cbase = (d // SUB) * (t_stride * SUB * TILE) + (d % SUB) * TILE
            pltpu.async_copy(
                ut_hbm.at[pl.ds(cbase, col_len)].at[uq_v], ubuf_v.at[d], sem
            )
            pltpu.async_copy(
                it_hbm.at[pl.ds(cbase, col_len)].at[iq_v], ibuf_v.at[d], sem
            )
        for d in range(D):
            pltpu.make_async_copy(
                ut_hbm.at[pl.ds(0, bpw)], ubuf_v.at[d], sem
            ).wait()
            pltpu.make_async_copy(
                it_hbm.at[pl.ds(0, bpw)], ibuf_v.at[d], sem
            ).wait()

        def body(j, carry):
            sl = pl.ds(j * L, L)
            acc = jnp.zeros((L,), jnp.float32)
            for d in range(D):
                acc = acc + ubuf_v[d, sl] * ibuf_v[d, sl]
            out_v[sl] = acc
            return carry

        lax.fori_loop(0, bpw // L, body, 0)

        pltpu.sync_copy(out_v, out_hbm.at[pl.ds(base, bpw)])

    def tail_tiles(t):
        tl = t.T[:, full_tiles * TILE:N - 1]
        return jnp.pad(tl, ((0, 0), (0, TILE - tail))).reshape(
            n_dd, SUB, TILE)

    u3, i3 = rebase_kernel(
        user_table.T, item_table.T,
        tail_tiles(user_table), tail_tiles(item_table),
    )
    return mf_kernel(user, item, u3.reshape(flat_len), i3.reshape(flat_len))
